# baseline (speedup 1.0000x reference)
import functools

import jax
import jax.numpy as jnp
import numpy as np
from jax import lax
from jax.experimental import pallas as pl
from jax.experimental.pallas import tpu as pltpu
from jax.experimental.pallas import tpu_sc as plsc

D_MODEL = 512
N_LEVELS = 4
N_HEADS = 16
N_POINTS = 4
SPATIAL = [(64, 64), (32, 32), (16, 16), (8, 8)]
BATCH = 2
LQ = sum(h * w for h, w in SPATIAL)
ROWS = BATCH * LQ
MLP = N_HEADS * N_LEVELS * N_POINTS
TERMS = MLP * 4

HROWS = LQ
BR = 640
NBLK = ROWS // BR
BR1 = 544
NBLK1 = HROWS // BR1

_NC, _NS = 2, 16
_NW = _NC * _NS
_QPW = HROWS // _NW
_CH = TERMS
_QC = 2
_NCH = _QPW // _QC


def _stage1_kernel(val_ref, q_ref, rpx_ref, rpy_ref, Wv_ref, bv_ref,
                   Wo_ref, bo_ref, Wa_ref, ba_ref, GE_ref,
                   v2_ref, idx_ref, wt_ref, rec_ref):
    i = pl.program_id(0)
    v2_ref[...] = (jnp.dot(val_ref[...], Wv_ref[...],
                           preferred_element_type=jnp.float32)
                   + bv_ref[...]).astype(jnp.bfloat16)
    q = q_ref[...]
    offa = jnp.dot(q, Wo_ref[...], preferred_element_type=jnp.float32) + bo_ref[...]
    offx = offa[:, :MLP]
    offy = offa[:, MLP:]
    logits = jnp.dot(q, Wa_ref[...], preferred_element_type=jnp.float32) + ba_ref[...]
    aw_e = jnp.exp(logits - jnp.max(logits, axis=1, keepdims=True))
    rec_ref[...] = 1.0 / jnp.dot(aw_e, GE_ref[...],
                                 preferred_element_type=jnp.float32)

    col = lax.broadcasted_iota(jnp.int32, (BR1, MLP), 1)
    mcol = col // 16
    lvl = (col % 16) // 4
    Wi = jnp.where(lvl == 0, 64, jnp.where(lvl == 1, 32, jnp.where(lvl == 2, 16, 8)))
    start = jnp.where(lvl == 0, 0, jnp.where(lvl == 1, 4096, jnp.where(lvl == 2, 5120, 5376)))
    Wf = Wi.astype(jnp.float32)

    tx = rpx_ref[...] + offx / Wf
    ty = rpy_ref[...] + offy / Wf
    gx = 2.0 * tx - 1.0
    gy = 2.0 * ty - 1.0
    x = ((gx + 1.0) * Wf - 1.0) * 0.5
    y = ((gy + 1.0) * Wf - 1.0) * 0.5
    x0 = jnp.floor(x)
    y0 = jnp.floor(y)
    fx = x - x0
    fy = y - y0

    idx_parts, wt_parts = [], []
    for cy in (0, 1):
        for cx in (0, 1):
            xi = x0 + cx
            yi = y0 + cy
            valid = (xi >= 0) & (xi <= Wf - 1) & (yi >= 0) & (yi <= Wf - 1)
            xc = jnp.clip(xi, 0, Wf - 1).astype(jnp.int32)
            yc = jnp.clip(yi, 0, Wf - 1).astype(jnp.int32)
            wxy = (fx if cx else (1.0 - fx)) * (fy if cy else (1.0 - fy))
            wt_parts.append(jnp.where(valid, wxy, 0.0) * aw_e)
            idx_parts.append((start + yc * Wi + xc) * N_HEADS + mcol)
    for p in range(8):
        part = idx_parts[p // 2]
        idx_ref[p] = part[:, (p % 2) * 128:(p % 2) * 128 + 128]
        wpart = wt_parts[p // 2]
        wt_ref[p] = wpart[:, (p % 2) * 128:(p % 2) * 128 + 128]


def _stage1(value2, query2, rpx, rpy, W_value, b_value, W_off_p, b_off_p,
            W_attn, b_attn, GE):
    d = D_MODEL
    return pl.pallas_call(
        _stage1_kernel,
        grid=(NBLK1,),
        in_specs=[
            pl.BlockSpec((BR1, d), lambda i: (i, 0)),
            pl.BlockSpec((BR1, d), lambda i: (i, 0)),
            pl.BlockSpec((BR1, MLP), lambda i: (i, 0)),
            pl.BlockSpec((BR1, MLP), lambda i: (i, 0)),
            pl.BlockSpec((d, d), lambda i: (0, 0)),
            pl.BlockSpec((d,), lambda i: (0,)),
            pl.BlockSpec((d, d), lambda i: (0, 0)),
            pl.BlockSpec((d,), lambda i: (0,)),
            pl.BlockSpec((d, MLP), lambda i: (0, 0)),
            pl.BlockSpec((MLP,), lambda i: (0,)),
            pl.BlockSpec((MLP, d), lambda i: (0, 0)),
        ],
        out_specs=[
            pl.BlockSpec((BR1, d), lambda i: (i, 0)),
            pl.BlockSpec((8, BR1, 128), lambda i: (0, i, 0)),
            pl.BlockSpec((8, BR1, 128), lambda i: (0, i, 0)),
            pl.BlockSpec((BR1, d), lambda i: (i, 0)),
        ],
        out_shape=[
            jax.ShapeDtypeStruct((HROWS, d), jnp.bfloat16),
            jax.ShapeDtypeStruct((8, HROWS, 128), jnp.int32),
            jax.ShapeDtypeStruct((8, HROWS, 128), jnp.float32),
            jax.ShapeDtypeStruct((HROWS, d), jnp.float32),
        ],
    )(value2, query2, rpx, rpy, W_value, b_value, W_off_p, b_off_p,
      W_attn, b_attn, GE)


@functools.cache
def _build_sc_sample():
    mesh = plsc.VectorSubcoreMesh(core_axis_name="c", subcore_axis_name="s",
                                  num_cores=_NC, num_subcores=_NS)
    return functools.partial(
        pl.kernel, mesh=mesh,
        compiler_params=pltpu.CompilerParams(use_tc_tiling_on_sc=False,
                                             needs_layout_passes=False),
        out_type=jax.ShapeDtypeStruct((HROWS * D_MODEL,), jnp.float32),
        scratch_types=[
            pltpu.VMEM((_QC * 8, 128), jnp.int32),
            pltpu.VMEM((_QC * 8, 128), jnp.float32),
            pltpu.VMEM((_QC * _CH, 32), jnp.bfloat16),
            pltpu.VMEM((_QC * D_MODEL,), jnp.float32),
            pltpu.VMEM((_QC * 8, 128), jnp.int32),
            pltpu.VMEM((_QC * 8, 128), jnp.float32),
            pltpu.VMEM((_QC * _CH, 32), jnp.bfloat16),
            pltpu.VMEM((_QC * D_MODEL,), jnp.float32),
        ] + [pltpu.SemaphoreType.DMA] * 8,
    )(_sc_sample_body)


def _sc_sample_body(table, idxf, wtf, outf,
                    idx_v0, wt_v0, rows_v0, out_v0,
                    idx_v1, wt_v1, rows_v1, out_v1,
                    si0, sg0, sw0, so0, si1, sg1, sw1, so1):
    wid = lax.axis_index("s") * _NC + lax.axis_index("c")
    qbase = wid * _QPW
    bufs = ((idx_v0, wt_v0, rows_v0, out_v0, si0, sg0, sw0, so0),
            (idx_v1, wt_v1, rows_v1, out_v1, si1, sg1, sw1, so1))

    def start_idx(k, buf):
        idx_v, _, _, _, si, _, _, _ = buf
        for qq in range(_QC):
            for p in range(_CH // 128):
                pltpu.async_copy(idxf.at[p, qbase + k * _QC + qq],
                                 idx_v.at[qq * 8 + p], si)

    def start_wt(k, buf):
        _, wt_v, _, _, _, _, sw, _ = buf
        for qq in range(_QC):
            for p in range(_CH // 128):
                pltpu.async_copy(wtf.at[p, qbase + k * _QC + qq],
                                 wt_v.at[qq * 8 + p], sw)

    def issue_gathers(buf):
        idx_v, _, rows_v, _, si, sg, _, _ = buf
        pltpu.make_async_copy(idxf.at[0, pl.ds(0, _QC * 8)],
                              idx_v, si).wait()
        for j in range(_QC * 8):
            pltpu.async_copy(table.at[idx_v.at[j]],
                             rows_v.at[pl.ds(j * 128, 128)], sg)

    def wait_gathers(buf):
        _, _, rows_v, _, _, sg, _, _ = buf
        pltpu.make_async_copy(table.at[pl.ds(0, _QC * _CH)], rows_v, sg).wait()

    def compute(k, buf):
        _, wt_v, rows_v, out_v, _, sg, sw, so = buf
        pltpu.make_async_copy(wtf.at[0, pl.ds(0, _QC * 8)], wt_v, sw).wait()

        @pl.when(k >= 2)
        def _():
            pltpu.make_async_copy(out_v, outf.at[pl.ds(0, _QC * D_MODEL)],
                                  so).wait()

        for qq in range(_QC):
            def mbody(m, _, qq=qq):
                base_m = qq * _CH + m * 16
                accs = []
                for cc in range(4):
                    base = base_m + cc * MLP
                    wt16 = wt_v[qq * 8 + cc * 2 + m // 8,
                                pl.ds((m % 8) * 16, 16)]
                    r0, r1 = plsc.unpack(rows_v[base, :],
                                         format=plsc.PackFormat.INTERLEAVED,
                                         preferred_element_type=jnp.float32)
                    w = wt16[0]
                    accs.append([w * r0, w * r1, wt16])
                for t in range(1, 16):
                    for cc in range(4):
                        base = base_m + cc * MLP
                        w = accs[cc][2][t]
                        r0, r1 = plsc.unpack(rows_v[base + t, :],
                                             format=plsc.PackFormat.INTERLEAVED,
                                             preferred_element_type=jnp.float32)
                        accs[cc][0] = accs[cc][0] + w * r0
                        accs[cc][1] = accs[cc][1] + w * r1
                acc0 = (accs[0][0] + accs[1][0]) + (accs[2][0] + accs[3][0])
                acc1 = (accs[0][1] + accs[1][1]) + (accs[2][1] + accs[3][1])
                o = qq * D_MODEL + m * 32
                out_v[pl.ds(o, 16)] = acc0
                out_v[pl.ds(o + 16, 16)] = acc1
                return 0

            lax.fori_loop(0, N_HEADS, mbody, 0)
        pltpu.async_copy(
            out_v, outf.at[pl.ds((qbase + k * _QC) * D_MODEL,
                                 _QC * D_MODEL)], so)

    start_idx(0, bufs[0])
    start_idx(1, bufs[1])
    start_wt(0, bufs[0])
    start_wt(1, bufs[1])
    issue_gathers(bufs[0])

    def pair(i, _):
        for b in range(2):
            k = 2 * i + b

            @pl.when(k + 1 < _NCH)
            def _():
                issue_gathers(bufs[1 - b])
            wait_gathers(bufs[b])

            @pl.when(k + 2 < _NCH)
            def _():
                start_idx(k + 2, bufs[b])
            compute(k, bufs[b])

            @pl.when(k + 2 < _NCH)
            def _():
                start_wt(k + 2, bufs[b])
        return 0

    lax.fori_loop(0, _NCH // 2, pair, 0)
    if _NCH % 2:
        wait_gathers(bufs[0])
        compute(_NCH - 1, bufs[0])
    pltpu.make_async_copy(out_v0, outf.at[pl.ds(0, _QC * D_MODEL)], so0).wait()
    pltpu.make_async_copy(out_v1, outf.at[pl.ds(0, _QC * D_MODEL)], so1).wait()


def _stage3_kernel(x_ref, rec_ref, w_ref, b_ref, o_ref):
    o_ref[...] = jnp.dot(x_ref[...] * rec_ref[...], w_ref[...],
                         preferred_element_type=jnp.float32) + b_ref[...]


def _stage3(x, rec, W_out, b_out):
    d = D_MODEL
    return pl.pallas_call(
        _stage3_kernel,
        grid=(NBLK,),
        in_specs=[
            pl.BlockSpec((BR, d), lambda i: (i, 0)),
            pl.BlockSpec((BR, d), lambda i: (i, 0)),
            pl.BlockSpec((d, d), lambda i: (0, 0)),
            pl.BlockSpec((d,), lambda i: (0,)),
        ],
        out_specs=pl.BlockSpec((BR, d), lambda i: (i, 0)),
        out_shape=jax.ShapeDtypeStruct((ROWS, d), jnp.float32),
    )(x, rec, W_out, b_out)


_GE = np.kron(np.eye(N_HEADS, dtype=np.float32),
              np.ones((N_LEVELS * N_POINTS, D_MODEL // N_HEADS), np.float32))

_VPERM = np.empty((D_MODEL,), np.int32)
for _m in range(N_HEADS):
    for _k in range(16):
        _VPERM[_m * 32 + 2 * _k] = _m * 32 + _k
        _VPERM[_m * 32 + 2 * _k + 1] = _m * 32 + 16 + _k


def kernel(query, reference_points, value, value_spatial_shapes,
           W_value, b_value, W_off, b_off, W_attn, b_attn, W_out, b_out):
    B, Lq, d = query.shape

    W_off_p = jnp.concatenate([W_off[:, 0::2], W_off[:, 1::2]], axis=1)
    b_off_p = jnp.concatenate([b_off[0::2], b_off[1::2]], axis=0)
    W_value_p = W_value[:, _VPERM]
    b_value_p = b_value[_VPERM]
    rp = reference_points.reshape(ROWS, N_LEVELS, 2)
    rp_lp_x = jnp.repeat(rp[:, :, 0], N_POINTS, axis=1)
    rp_lp_y = jnp.repeat(rp[:, :, 1], N_POINTS, axis=1)
    rpx = jnp.tile(rp_lp_x, (1, N_HEADS))
    rpy = jnp.tile(rp_lp_y, (1, N_HEADS))

    val2 = value.reshape(ROWS, d)
    qu2 = query.reshape(ROWS, d)
    GE = jnp.asarray(_GE)
    sc = _build_sc_sample()
    sc_outs, recs = [], []
    for h in range(BATCH):
        sl = slice(h * HROWS, (h + 1) * HROWS)
        v2, idx, wt, rec = _stage1(
            val2[sl], qu2[sl], rpx[sl], rpy[sl],
            W_value_p, b_value_p, W_off_p, b_off_p, W_attn, b_attn, GE)
        table = v2.reshape(HROWS * N_HEADS, d // N_HEADS)
        sc_outs.append(sc(table, idx, wt).reshape(HROWS, d))
        recs.append(rec)
    sc_out = jnp.concatenate(sc_outs, axis=0)
    rec = jnp.concatenate(recs, axis=0)
    final = _stage3(sc_out, rec, W_out, b_out)
    return final.reshape(B, Lq, d)


# speedup vs baseline: 1.0901x; 1.0901x over previous
import functools

import jax
import jax.numpy as jnp
import numpy as np
from jax import lax
from jax.experimental import pallas as pl
from jax.experimental.pallas import tpu as pltpu
from jax.experimental.pallas import tpu_sc as plsc

D_MODEL = 512
N_LEVELS = 4
N_HEADS = 16
N_POINTS = 4
SPATIAL = [(64, 64), (32, 32), (16, 16), (8, 8)]
BATCH = 2
LQ = sum(h * w for h, w in SPATIAL)
ROWS = BATCH * LQ
MLP = N_HEADS * N_LEVELS * N_POINTS
TERMS = MLP * 4

BR = 640
NBLK = ROWS // BR

_NC, _NS = 2, 16
_NW = _NC * _NS
_QPW = ROWS // _NW
_CH = TERMS
_QC = 2
_NCH = _QPW // _QC


def _stage1_kernel(val_ref, q_ref, rpx_ref, rpy_ref, Wv_ref, bv_ref,
                   Wo_ref, bo_ref, Wa_ref, ba_ref, GE_ref,
                   v2_ref, idx_ref, wt_ref, rec_ref):
    i = pl.program_id(0)
    v2_ref[...] = (jnp.dot(val_ref[...], Wv_ref[...],
                           preferred_element_type=jnp.float32)
                   + bv_ref[...]).astype(jnp.bfloat16)
    q = q_ref[...]
    offa = jnp.dot(q, Wo_ref[...], preferred_element_type=jnp.float32) + bo_ref[...]
    offx = offa[:, :MLP]
    offy = offa[:, MLP:]
    logits = jnp.dot(q, Wa_ref[...], preferred_element_type=jnp.float32) + ba_ref[...]
    aw_e = jnp.exp(logits - jnp.max(logits, axis=1, keepdims=True))
    rec_ref[...] = 1.0 / jnp.dot(aw_e, GE_ref[...],
                                 preferred_element_type=jnp.float32)

    col = lax.broadcasted_iota(jnp.int32, (BR, MLP), 1)
    mcol = col // 16
    lvl = (col % 16) // 4
    Wi = jnp.where(lvl == 0, 64, jnp.where(lvl == 1, 32, jnp.where(lvl == 2, 16, 8)))
    start = jnp.where(lvl == 0, 0, jnp.where(lvl == 1, 4096, jnp.where(lvl == 2, 5120, 5376)))
    Wf = Wi.astype(jnp.float32)

    tx = rpx_ref[...] + offx / Wf
    ty = rpy_ref[...] + offy / Wf
    gx = 2.0 * tx - 1.0
    gy = 2.0 * ty - 1.0
    x = ((gx + 1.0) * Wf - 1.0) * 0.5
    y = ((gy + 1.0) * Wf - 1.0) * 0.5
    x0 = jnp.floor(x)
    y0 = jnp.floor(y)
    fx = x - x0
    fy = y - y0

    row = i * BR + lax.broadcasted_iota(jnp.int32, (BR, MLP), 0)
    base_b = jnp.where(row >= LQ, LQ, 0)

    idx_parts, wt_parts = [], []
    for cy in (0, 1):
        for cx in (0, 1):
            xi = x0 + cx
            yi = y0 + cy
            valid = (xi >= 0) & (xi <= Wf - 1) & (yi >= 0) & (yi <= Wf - 1)
            xc = jnp.clip(xi, 0, Wf - 1).astype(jnp.int32)
            yc = jnp.clip(yi, 0, Wf - 1).astype(jnp.int32)
            wxy = (fx if cx else (1.0 - fx)) * (fy if cy else (1.0 - fy))
            wt_parts.append(jnp.where(valid, wxy, 0.0) * aw_e)
            idx_parts.append((base_b + start + yc * Wi + xc) * N_HEADS + mcol)
    for p in range(8):
        part = idx_parts[p // 2]
        idx_ref[p] = part[:, (p % 2) * 128:(p % 2) * 128 + 128]
        wpart = wt_parts[p // 2]
        wt_ref[p] = wpart[:, (p % 2) * 128:(p % 2) * 128 + 128]


def _stage1(value2, query2, rpx, rpy, W_value, b_value, W_off_p, b_off_p,
            W_attn, b_attn, GE):
    d = D_MODEL
    return pl.pallas_call(
        _stage1_kernel,
        grid=(NBLK,),
        in_specs=[
            pl.BlockSpec((BR, d), lambda i: (i, 0)),
            pl.BlockSpec((BR, d), lambda i: (i, 0)),
            pl.BlockSpec((BR, MLP), lambda i: (i, 0)),
            pl.BlockSpec((BR, MLP), lambda i: (i, 0)),
            pl.BlockSpec((d, d), lambda i: (0, 0)),
            pl.BlockSpec((d,), lambda i: (0,)),
            pl.BlockSpec((d, d), lambda i: (0, 0)),
            pl.BlockSpec((d,), lambda i: (0,)),
            pl.BlockSpec((d, MLP), lambda i: (0, 0)),
            pl.BlockSpec((MLP,), lambda i: (0,)),
            pl.BlockSpec((MLP, d), lambda i: (0, 0)),
        ],
        out_specs=[
            pl.BlockSpec((BR, d), lambda i: (i, 0)),
            pl.BlockSpec((8, BR, 128), lambda i: (0, i, 0)),
            pl.BlockSpec((8, BR, 128), lambda i: (0, i, 0)),
            pl.BlockSpec((BR, d), lambda i: (i, 0)),
        ],
        out_shape=[
            jax.ShapeDtypeStruct((ROWS, d), jnp.bfloat16),
            jax.ShapeDtypeStruct((8, ROWS, 128), jnp.int32),
            jax.ShapeDtypeStruct((8, ROWS, 128), jnp.float32),
            jax.ShapeDtypeStruct((ROWS, d), jnp.float32),
        ],
    )(value2, query2, rpx, rpy, W_value, b_value, W_off_p, b_off_p,
      W_attn, b_attn, GE)


@functools.cache
def _build_sc_sample():
    mesh = plsc.VectorSubcoreMesh(core_axis_name="c", subcore_axis_name="s",
                                  num_cores=_NC, num_subcores=_NS)
    return functools.partial(
        pl.kernel, mesh=mesh,
        compiler_params=pltpu.CompilerParams(use_tc_tiling_on_sc=False,
                                             needs_layout_passes=False),
        out_type=jax.ShapeDtypeStruct((ROWS * D_MODEL,), jnp.float32),
        scratch_types=[
            pltpu.VMEM((_QC * 8, 128), jnp.int32),
            pltpu.VMEM((_QC * 8, 128), jnp.float32),
            pltpu.VMEM((_QC * _CH, 32), jnp.bfloat16),
            pltpu.VMEM((_QC * D_MODEL,), jnp.float32),
            pltpu.VMEM((_QC * 8, 128), jnp.int32),
            pltpu.VMEM((_QC * 8, 128), jnp.float32),
            pltpu.VMEM((_QC * _CH, 32), jnp.bfloat16),
            pltpu.VMEM((_QC * D_MODEL,), jnp.float32),
        ] + [pltpu.SemaphoreType.DMA] * 8,
    )(_sc_sample_body)


def _sc_sample_body(table, idxf, wtf, outf,
                    idx_v0, wt_v0, rows_v0, out_v0,
                    idx_v1, wt_v1, rows_v1, out_v1,
                    si0, sg0, sw0, so0, si1, sg1, sw1, so1):
    wid = lax.axis_index("s") * _NC + lax.axis_index("c")
    qbase = wid * _QPW
    bufs = ((idx_v0, wt_v0, rows_v0, out_v0, si0, sg0, sw0, so0),
            (idx_v1, wt_v1, rows_v1, out_v1, si1, sg1, sw1, so1))

    def start_idx(k, buf):
        idx_v, _, _, _, si, _, _, _ = buf
        for qq in range(_QC):
            for p in range(_CH // 128):
                pltpu.async_copy(idxf.at[p, qbase + k * _QC + qq],
                                 idx_v.at[qq * 8 + p], si)

    def start_wt(k, buf):
        _, wt_v, _, _, _, _, sw, _ = buf
        for qq in range(_QC):
            for p in range(_CH // 128):
                pltpu.async_copy(wtf.at[p, qbase + k * _QC + qq],
                                 wt_v.at[qq * 8 + p], sw)

    def issue_gathers(buf):
        idx_v, _, rows_v, _, si, sg, _, _ = buf
        pltpu.make_async_copy(idxf.at[0, pl.ds(0, _QC * 8)],
                              idx_v, si).wait()
        for j in range(_QC * 8):
            pltpu.async_copy(table.at[idx_v.at[j]],
                             rows_v.at[pl.ds(j * 128, 128)], sg)

    def wait_gathers(buf):
        _, _, rows_v, _, _, sg, _, _ = buf
        pltpu.make_async_copy(table.at[pl.ds(0, _QC * _CH)], rows_v, sg).wait()

    def compute(k, buf):
        _, wt_v, rows_v, out_v, _, sg, sw, so = buf
        pltpu.make_async_copy(wtf.at[0, pl.ds(0, _QC * 8)], wt_v, sw).wait()

        @pl.when(k >= 2)
        def _():
            pltpu.make_async_copy(out_v, outf.at[pl.ds(0, _QC * D_MODEL)],
                                  so).wait()

        for qq in range(_QC):
            def mbody(m, _, qq=qq):
                base_m = qq * _CH + m * 16
                accs = []
                for cc in range(4):
                    base = base_m + cc * MLP
                    wt16 = wt_v[qq * 8 + cc * 2 + m // 8,
                                pl.ds((m % 8) * 16, 16)]
                    r0, r1 = plsc.unpack(rows_v[base, :],
                                         format=plsc.PackFormat.INTERLEAVED,
                                         preferred_element_type=jnp.float32)
                    w = wt16[0]
                    accs.append([w * r0, w * r1, wt16])
                for t in range(1, 16):
                    for cc in range(4):
                        base = base_m + cc * MLP
                        w = accs[cc][2][t]
                        r0, r1 = plsc.unpack(rows_v[base + t, :],
                                             format=plsc.PackFormat.INTERLEAVED,
                                             preferred_element_type=jnp.float32)
                        accs[cc][0] = accs[cc][0] + w * r0
                        accs[cc][1] = accs[cc][1] + w * r1
                acc0 = (accs[0][0] + accs[1][0]) + (accs[2][0] + accs[3][0])
                acc1 = (accs[0][1] + accs[1][1]) + (accs[2][1] + accs[3][1])
                o = qq * D_MODEL + m * 32
                out_v[pl.ds(o, 16)] = acc0
                out_v[pl.ds(o + 16, 16)] = acc1
                return 0

            lax.fori_loop(0, N_HEADS, mbody, 0)
        pltpu.async_copy(
            out_v, outf.at[pl.ds((qbase + k * _QC) * D_MODEL,
                                 _QC * D_MODEL)], so)

    start_idx(0, bufs[0])
    start_idx(1, bufs[1])
    start_wt(0, bufs[0])
    start_wt(1, bufs[1])
    issue_gathers(bufs[0])

    def pair(i, _):
        for b in range(2):
            k = 2 * i + b

            @pl.when(k + 1 < _NCH)
            def _():
                issue_gathers(bufs[1 - b])
            wait_gathers(bufs[b])

            @pl.when(k + 2 < _NCH)
            def _():
                start_idx(k + 2, bufs[b])
            compute(k, bufs[b])

            @pl.when(k + 2 < _NCH)
            def _():
                start_wt(k + 2, bufs[b])
        return 0

    lax.fori_loop(0, _NCH // 2, pair, 0)
    pltpu.make_async_copy(out_v0, outf.at[pl.ds(0, _QC * D_MODEL)], so0).wait()
    pltpu.make_async_copy(out_v1, outf.at[pl.ds(0, _QC * D_MODEL)], so1).wait()


def _stage3_kernel(x_ref, rec_ref, w_ref, b_ref, o_ref):
    o_ref[...] = jnp.dot(x_ref[...] * rec_ref[...], w_ref[...],
                         preferred_element_type=jnp.float32) + b_ref[...]


def _stage3(x, rec, W_out, b_out):
    d = D_MODEL
    return pl.pallas_call(
        _stage3_kernel,
        grid=(NBLK,),
        in_specs=[
            pl.BlockSpec((BR, d), lambda i: (i, 0)),
            pl.BlockSpec((BR, d), lambda i: (i, 0)),
            pl.BlockSpec((d, d), lambda i: (0, 0)),
            pl.BlockSpec((d,), lambda i: (0,)),
        ],
        out_specs=pl.BlockSpec((BR, d), lambda i: (i, 0)),
        out_shape=jax.ShapeDtypeStruct((ROWS, d), jnp.float32),
    )(x, rec, W_out, b_out)


_GE = np.kron(np.eye(N_HEADS, dtype=np.float32),
              np.ones((N_LEVELS * N_POINTS, D_MODEL // N_HEADS), np.float32))

_VPERM = np.empty((D_MODEL,), np.int32)
for _m in range(N_HEADS):
    for _k in range(16):
        _VPERM[_m * 32 + 2 * _k] = _m * 32 + _k
        _VPERM[_m * 32 + 2 * _k + 1] = _m * 32 + 16 + _k


def kernel(query, reference_points, value, value_spatial_shapes,
           W_value, b_value, W_off, b_off, W_attn, b_attn, W_out, b_out):
    B, Lq, d = query.shape

    W_off_p = jnp.concatenate([W_off[:, 0::2], W_off[:, 1::2]], axis=1)
    b_off_p = jnp.concatenate([b_off[0::2], b_off[1::2]], axis=0)
    W_value_p = W_value[:, _VPERM]
    b_value_p = b_value[_VPERM]
    rp = reference_points.reshape(ROWS, N_LEVELS, 2)
    rp_lp_x = jnp.repeat(rp[:, :, 0], N_POINTS, axis=1)
    rp_lp_y = jnp.repeat(rp[:, :, 1], N_POINTS, axis=1)
    rpx = jnp.tile(rp_lp_x, (1, N_HEADS))
    rpy = jnp.tile(rp_lp_y, (1, N_HEADS))

    v2, idx, wt, rec = _stage1(
        value.reshape(ROWS, d), query.reshape(ROWS, d), rpx, rpy,
        W_value_p, b_value_p, W_off_p, b_off_p, W_attn, b_attn,
        jnp.asarray(_GE))

    table = v2.reshape(BATCH * Lq * N_HEADS, d // N_HEADS)
    sc_out = _build_sc_sample()(table, idx, wt).reshape(ROWS, d)
    final = _stage3(sc_out, rec, W_out, b_out)
    return final.reshape(B, Lq, d)
